# Initial kernel scaffold; baseline (speedup 1.0000x reference)
import jax
import jax.numpy as jnp
from jax.experimental import pallas as pl


def kernel(h, cb0, cb1, cb2, cb3):
    raise NotImplementedError("write your pallas kernel here")


# speedup vs baseline: 1.1731x; 1.1731x over previous
import functools

import jax
import jax.numpy as jnp
from jax.experimental import pallas as pl

H_D = 256
N_LEVELS = 4
N_TOKENS = 8192
BLK = 512


def _vq_body(h_ref, cb0_ref, cb1_ref, cb2_ref, cb3_ref,
             zq_ref, sid_ref, r_ref, q_ref):
    r = h_ref[...]
    zq = jnp.zeros_like(r)
    for i, cb_ref in enumerate((cb0_ref, cb1_ref, cb2_ref, cb3_ref)):
        cb = cb_ref[...]
        r2 = jnp.sum(r * r, axis=1, keepdims=True)
        c2 = jnp.sum(cb * cb, axis=1)[None, :]
        cross = jax.lax.dot_general(
            r.astype(jnp.bfloat16), cb.astype(jnp.bfloat16),
            (((1,), (1,)), ((), ())),
            preferred_element_type=jnp.float32)
        d = r2 + c2 - 2.0 * cross
        if cb.shape[0] > 1024:
            d1, d2 = d[:, :1024], d[:, 1024:]
            m1 = jnp.min(d1, axis=1)
            i1 = jnp.argmin(d1, axis=1).astype(jnp.int32)
            m2 = jnp.min(d2, axis=1)
            i2 = jnp.argmin(d2, axis=1).astype(jnp.int32) + 1024
            m1q = m1.astype(jnp.bfloat16).astype(jnp.float32)
            idx = jnp.where(m2 < m1q, i2, i1)
        else:
            idx = jnp.argmin(d, axis=1).astype(jnp.int32)
        onehot = (jax.lax.broadcasted_iota(jnp.int32, d.shape, 1)
                  == idx[:, None]).astype(jnp.float32)
        q = jax.lax.dot_general(
            onehot, cb, (((1,), (0,)), ((), ())),
            precision=jax.lax.Precision.HIGHEST,
            preferred_element_type=jnp.float32)
        sid_ref[:, i] = idx
        r_ref[:, i, :] = r
        q_ref[:, i, :] = q
        zq = zq + q
        r = r - q
    zq_ref[...] = zq


@jax.jit
def kernel(h, cb0, cb1, cb2, cb3):
    n = h.shape[0]
    grid = (n // BLK,)
    cb_spec = lambda k: pl.BlockSpec((k, H_D), lambda i: (0, 0))
    out_shapes = (
        jax.ShapeDtypeStruct((n, H_D), jnp.float32),
        jax.ShapeDtypeStruct((n, N_LEVELS), jnp.int32),
        jax.ShapeDtypeStruct((n, N_LEVELS, H_D), jnp.float32),
        jax.ShapeDtypeStruct((n, N_LEVELS, H_D), jnp.float32),
    )
    z_q, sids, r_list, q_list = pl.pallas_call(
        _vq_body,
        grid=grid,
        in_specs=[
            pl.BlockSpec((BLK, H_D), lambda i: (i, 0)),
            cb_spec(cb0.shape[0]),
            cb_spec(cb1.shape[0]),
            cb_spec(cb2.shape[0]),
            cb_spec(cb3.shape[0]),
        ],
        out_specs=(
            pl.BlockSpec((BLK, H_D), lambda i: (i, 0)),
            pl.BlockSpec((BLK, N_LEVELS), lambda i: (i, 0)),
            pl.BlockSpec((BLK, N_LEVELS, H_D), lambda i: (i, 0, 0)),
            pl.BlockSpec((BLK, N_LEVELS, H_D), lambda i: (i, 0, 0)),
        ),
        out_shape=out_shapes,
    )(h, cb0, cb1, cb2, cb3)
    return (z_q, sids, r_list, q_list)
